# baseline (speedup 1.0000x reference)
import functools

import jax
import jax.numpy as jnp
from jax import lax
from jax.experimental import pallas as pl
from jax.experimental.pallas import tpu as pltpu
from jax.experimental.pallas import tpu_sc as plsc

EMB = 64
CTX = 50
NCLS = 6
CB = 4
NBUF = 2
TC_VB = 4096


QUARTER = 262144


def _rtne16(t):
    xb = lax.bitcast_convert_type(t, jnp.uint32)
    return xb + jnp.uint32(0x7FFF) + ((xb >> 16) & jnp.uint32(1))


def _transpose_body(r0, r1, r2, r3, out_ref):
    h = EMB // 2
    ident = (lax.broadcasted_iota(jnp.int32, (h, h), 0)
             == lax.broadcasted_iota(jnp.int32, (h, h), 1)).astype(jnp.float32)
    dn = (((0,), (0,)), ((), ()))
    packs = []
    for ref in (r0, r1, r2, r3):
        x = ref[...]
        tlo = lax.dot_general(x[0:h, :], ident, dn,
                              preferred_element_type=jnp.float32)
        thi = lax.dot_general(x[h:EMB, :], ident, dn,
                              preferred_element_type=jnp.float32)
        word = (_rtne16(tlo) >> 16) | (_rtne16(thi) & jnp.uint32(0xFFFF0000))
        packs.append(lax.bitcast_convert_type(word, jnp.int32))
    out_ref[...] = jnp.concatenate(packs, axis=1)


def _repack_table(table_t):
    emb, vocab = table_t.shape
    steps = QUARTER // TC_VB
    max_blk = (vocab - 1) // TC_VB

    def make_map(q):
        return lambda j: (0, jnp.minimum(q * steps + j, max_blk))

    return pl.pallas_call(
        _transpose_body,
        grid=(steps,),
        in_specs=[pl.BlockSpec((emb, TC_VB), make_map(q)) for q in range(4)],
        out_specs=pl.BlockSpec((TC_VB, 2 * emb), lambda j: (j, 0)),
        out_shape=jax.ShapeDtypeStruct((QUARTER, 2 * emb), jnp.int32),
    )(table_t, table_t, table_t, table_t)


def _make_pool_kernel(batch):
    info = plsc.get_sparse_core_info()
    nw = info.num_cores * info.num_subcores
    bpw = batch // nw
    rows = CB * CTX
    nchunk = bpw // CB
    mesh = plsc.VectorSubcoreMesh(core_axis_name="c", subcore_axis_name="s")

    @functools.partial(
        pl.kernel,
        out_type=jax.ShapeDtypeStruct((batch, EMB), jnp.float32),
        mesh=mesh,
        scratch_types=[
            pltpu.VMEM((bpw * CTX,), jnp.int32),
            pltpu.VMEM((NBUF, rows, EMB // 2), jnp.int32),
            pltpu.VMEM((bpw, EMB), jnp.float32),
        ] + [pltpu.SemaphoreType.DMA] * NBUF,
        compiler_params=pltpu.CompilerParams(use_tc_tiling_on_sc=False),
    )
    def pool(table_hbm, idx_hbm, out_hbm, idx_v, rows_v, pooled_v, *sems):
        wid = lax.axis_index("s") * info.num_cores + lax.axis_index("c")
        base = wid * bpw
        pltpu.sync_copy(idx_hbm.at[pl.ds(base * CTX, bpw * CTX)], idx_v)

        def gather(c, b):
            return pltpu.make_async_copy(
                table_hbm.at[idx_v.at[pl.ds(c * rows, rows)]],
                rows_v.at[b], sems[b])

        def halves(b, row):
            out = []
            for h in range(2):
                w = rows_v[b, row, pl.ds(16 * h, 16)]
                out.append((lax.bitcast_convert_type(w << 16, jnp.float32),
                            lax.bitcast_convert_type(w & jnp.int32(-65536),
                                                     jnp.float32)))
            (a0, b0), (a1, b1) = out
            return a0, a1, b0, b1

        for b in range(NBUF):
            gather(b, b).start()

        @pl.loop(0, nchunk // NBUF)
        def _group(i):
            for b in range(NBUF):
                c = NBUF * i + b
                gather(c, b).wait()
                for e in range(CB):
                    accs = list(halves(b, e * CTX))
                    for r in range(1, CTX):
                        hs = halves(b, e * CTX + r)
                        for k in range(4):
                            accs[k] = accs[k] + hs[k]
                    for k in range(4):
                        pooled_v[c * CB + e, pl.ds(16 * k, 16)] = accs[k]

                @pl.when(c + NBUF < nchunk)
                def _prefetch():
                    gather(c + NBUF, b).start()

        pltpu.sync_copy(pooled_v, out_hbm.at[pl.ds(base, bpw)])

    return pool


def _linear_body(p_ref, wt_ref, b_ref, o_ref):
    acc = jnp.dot(p_ref[...], wt_ref[...], preferred_element_type=jnp.float32)
    o_ref[...] = jax.nn.sigmoid(acc + b_ref[...])


def _linear(pooled, wt8, b8):
    batch = pooled.shape[0]
    blk = 2048
    grid = batch // blk
    return pl.pallas_call(
        _linear_body,
        grid=(grid,),
        in_specs=[
            pl.BlockSpec((blk, EMB), lambda i: (i, 0)),
            pl.BlockSpec((EMB, 8), lambda i: (0, 0)),
            pl.BlockSpec((1, 8), lambda i: (0, 0)),
        ],
        out_specs=pl.BlockSpec((blk, 8), lambda i: (i, 0)),
        out_shape=jax.ShapeDtypeStruct((batch, 8), jnp.float32),
    )(pooled, wt8, b8)


def kernel(inputs, table, W, b):
    ctx, batch = inputs.shape
    vocab = table.shape[0]
    idx_flat = inputs.T.reshape(-1).astype(jnp.int32)
    idx_flat = 4 * (idx_flat % QUARTER) + idx_flat // QUARTER
    table_pk = _repack_table(table.T).reshape(4 * QUARTER, EMB // 2)
    pooled = _make_pool_kernel(batch)(table_pk, idx_flat)
    wt8 = jnp.zeros((EMB, 8), jnp.float32).at[:, :NCLS].set(W.T)
    b8 = jnp.zeros((1, 8), jnp.float32).at[0, :NCLS].set(b)
    out8 = _linear(pooled, wt8, b8)
    return out8[:, :NCLS]


# speedup vs baseline: 1.1491x; 1.1491x over previous
import functools

import jax
import jax.numpy as jnp
from jax import lax
from jax.experimental import pallas as pl
from jax.experimental.pallas import tpu as pltpu
from jax.experimental.pallas import tpu_sc as plsc

EMB = 64
CTX = 50
NCLS = 6
CB = 4
NBUF = 2
TC_VB = 4096


QUARTER = 262144


def _rtne16(t):
    xb = lax.bitcast_convert_type(t, jnp.uint32)
    return xb + jnp.uint32(0x7FFF) + ((xb >> 16) & jnp.uint32(1))


def _transpose_body(r0, r1, r2, r3, out_ref):
    h = EMB // 2
    ident = (lax.broadcasted_iota(jnp.int32, (EMB, EMB), 0)
             == lax.broadcasted_iota(jnp.int32, (EMB, EMB), 1)).astype(jnp.float32)
    dn = (((0,), (0,)), ((), ()))
    packs = []
    for ref in (r0, r1, r2, r3):
        t = lax.dot_general(ref[...], ident, dn,
                            preferred_element_type=jnp.float32)
        r = _rtne16(t)
        word = (r[:, 0:h] >> 16) | (r[:, h:EMB] & jnp.uint32(0xFFFF0000))
        packs.append(lax.bitcast_convert_type(word, jnp.int32))
    out_ref[...] = jnp.concatenate(packs, axis=1)


def _repack_table(table_t):
    emb, vocab = table_t.shape
    steps = QUARTER // TC_VB
    max_blk = (vocab - 1) // TC_VB

    def make_map(q):
        return lambda j: (0, jnp.minimum(q * steps + j, max_blk))

    return pl.pallas_call(
        _transpose_body,
        grid=(steps,),
        in_specs=[pl.BlockSpec((emb, TC_VB), make_map(q)) for q in range(4)],
        out_specs=pl.BlockSpec((TC_VB, 2 * emb), lambda j: (j, 0)),
        out_shape=jax.ShapeDtypeStruct((QUARTER, 2 * emb), jnp.int32),
    )(table_t, table_t, table_t, table_t)


def _make_pool_kernel(batch):
    info = plsc.get_sparse_core_info()
    nw = info.num_cores * info.num_subcores
    bpw = batch // nw
    rows = CB * CTX
    nchunk = bpw // CB
    mesh = plsc.VectorSubcoreMesh(core_axis_name="c", subcore_axis_name="s")

    @functools.partial(
        pl.kernel,
        out_type=jax.ShapeDtypeStruct((batch, EMB), jnp.float32),
        mesh=mesh,
        scratch_types=[
            pltpu.VMEM((bpw * CTX,), jnp.int32),
            pltpu.VMEM((NBUF, rows, EMB // 2), jnp.int32),
            pltpu.VMEM((bpw, EMB), jnp.float32),
        ] + [pltpu.SemaphoreType.DMA] * NBUF,
        compiler_params=pltpu.CompilerParams(use_tc_tiling_on_sc=False),
    )
    def pool(table_hbm, idx_hbm, out_hbm, idx_v, rows_v, pooled_v, *sems):
        wid = lax.axis_index("s") * info.num_cores + lax.axis_index("c")
        base = wid * bpw
        pltpu.sync_copy(idx_hbm.at[pl.ds(base * CTX, bpw * CTX)], idx_v)

        def gather(c, b):
            return pltpu.make_async_copy(
                table_hbm.at[idx_v.at[pl.ds(c * rows, rows)]],
                rows_v.at[b], sems[b])

        def halves(b, row):
            out = []
            for h in range(2):
                w = rows_v[b, row, pl.ds(16 * h, 16)]
                out.append((lax.bitcast_convert_type(w << 16, jnp.float32),
                            lax.bitcast_convert_type(w & jnp.int32(-65536),
                                                     jnp.float32)))
            (a0, b0), (a1, b1) = out
            return a0, a1, b0, b1

        for b in range(NBUF):
            gather(b, b).start()

        @pl.loop(0, nchunk // NBUF)
        def _group(i):
            for b in range(NBUF):
                c = NBUF * i + b
                gather(c, b).wait()
                for e in range(CB):
                    accs = list(halves(b, e * CTX))
                    for r in range(1, CTX):
                        hs = halves(b, e * CTX + r)
                        for k in range(4):
                            accs[k] = accs[k] + hs[k]
                    for k in range(4):
                        pooled_v[c * CB + e, pl.ds(16 * k, 16)] = accs[k]

                @pl.when(c + NBUF < nchunk)
                def _prefetch():
                    gather(c + NBUF, b).start()

        pltpu.sync_copy(pooled_v, out_hbm.at[pl.ds(base, bpw)])

    return pool


def _linear_body(p_ref, wt_ref, b_ref, o_ref):
    acc = jnp.dot(p_ref[...], wt_ref[...], preferred_element_type=jnp.float32)
    o_ref[...] = jax.nn.sigmoid(acc + b_ref[...])


def _linear(pooled, wt8, b8):
    batch = pooled.shape[0]
    blk = 2048
    grid = batch // blk
    return pl.pallas_call(
        _linear_body,
        grid=(grid,),
        in_specs=[
            pl.BlockSpec((blk, EMB), lambda i: (i, 0)),
            pl.BlockSpec((EMB, 8), lambda i: (0, 0)),
            pl.BlockSpec((1, 8), lambda i: (0, 0)),
        ],
        out_specs=pl.BlockSpec((blk, 8), lambda i: (i, 0)),
        out_shape=jax.ShapeDtypeStruct((batch, 8), jnp.float32),
    )(pooled, wt8, b8)


def kernel(inputs, table, W, b):
    ctx, batch = inputs.shape
    vocab = table.shape[0]
    idx_flat = inputs.T.reshape(-1).astype(jnp.int32)
    idx_flat = 4 * (idx_flat % QUARTER) + idx_flat // QUARTER
    table_pk = _repack_table(table.T).reshape(4 * QUARTER, EMB // 2)
    pooled = _make_pool_kernel(batch)(table_pk, idx_flat)
    wt8 = jnp.zeros((EMB, 8), jnp.float32).at[:, :NCLS].set(W.T)
    b8 = jnp.zeros((1, 8), jnp.float32).at[0, :NCLS].set(b)
    out8 = _linear(pooled, wt8, b8)
    return out8[:, :NCLS]


# speedup vs baseline: 1.1537x; 1.0040x over previous
import functools

import jax
import jax.numpy as jnp
from jax import lax
from jax.experimental import pallas as pl
from jax.experimental.pallas import tpu as pltpu
from jax.experimental.pallas import tpu_sc as plsc

EMB = 64
CTX = 50
NCLS = 6
CB = 4
NBUF = 2
TC_VB = 4096


QUARTER = 262144


def _rtne16(t):
    xb = lax.bitcast_convert_type(t, jnp.uint32)
    return xb + jnp.uint32(0x7FFF) + ((xb >> 16) & jnp.uint32(1))


def _transpose_body(r0, r1, r2, r3, out_ref):
    h = EMB // 2
    ident = (lax.broadcasted_iota(jnp.int32, (EMB, EMB), 0)
             == lax.broadcasted_iota(jnp.int32, (EMB, EMB), 1)).astype(jnp.bfloat16)
    dn = (((0,), (0,)), ((), ()))
    packs = []
    for ref in (r0, r1, r2, r3):
        t = lax.dot_general(ref[...].astype(jnp.bfloat16), ident, dn,
                            preferred_element_type=jnp.float32)
        r = lax.bitcast_convert_type(t, jnp.uint32)
        word = (r[:, 0:h] >> 16) | r[:, h:EMB]
        packs.append(lax.bitcast_convert_type(word, jnp.int32))
    out_ref[...] = jnp.concatenate(packs, axis=1)


def _repack_table(table_t):
    emb, vocab = table_t.shape
    steps = QUARTER // TC_VB
    max_blk = (vocab - 1) // TC_VB

    def make_map(q):
        return lambda j: (0, jnp.minimum(q * steps + j, max_blk))

    return pl.pallas_call(
        _transpose_body,
        grid=(steps,),
        in_specs=[pl.BlockSpec((emb, TC_VB), make_map(q)) for q in range(4)],
        out_specs=pl.BlockSpec((TC_VB, 2 * emb), lambda j: (j, 0)),
        out_shape=jax.ShapeDtypeStruct((QUARTER, 2 * emb), jnp.int32),
    )(table_t, table_t, table_t, table_t)


def _make_pool_kernel(batch):
    info = plsc.get_sparse_core_info()
    nw = info.num_cores * info.num_subcores
    bpw = batch // nw
    rows = CB * CTX
    nchunk = bpw // CB
    mesh = plsc.VectorSubcoreMesh(core_axis_name="c", subcore_axis_name="s")

    @functools.partial(
        pl.kernel,
        out_type=jax.ShapeDtypeStruct((batch, EMB), jnp.float32),
        mesh=mesh,
        scratch_types=[
            pltpu.VMEM((bpw * CTX,), jnp.int32),
            pltpu.VMEM((NBUF, rows, EMB // 2), jnp.int32),
            pltpu.VMEM((bpw, EMB), jnp.float32),
        ] + [pltpu.SemaphoreType.DMA] * NBUF,
        compiler_params=pltpu.CompilerParams(use_tc_tiling_on_sc=False),
    )
    def pool(table_hbm, idx_hbm, out_hbm, idx_v, rows_v, pooled_v, *sems):
        wid = lax.axis_index("s") * info.num_cores + lax.axis_index("c")
        base = wid * bpw
        pltpu.sync_copy(idx_hbm.at[pl.ds(base * CTX, bpw * CTX)], idx_v)

        def gather(c, b):
            return pltpu.make_async_copy(
                table_hbm.at[idx_v.at[pl.ds(c * rows, rows)]],
                rows_v.at[b], sems[b])

        def halves(b, row):
            out = []
            for h in range(2):
                w = rows_v[b, row, pl.ds(16 * h, 16)]
                out.append((lax.bitcast_convert_type(w << 16, jnp.float32),
                            lax.bitcast_convert_type(w & jnp.int32(-65536),
                                                     jnp.float32)))
            (a0, b0), (a1, b1) = out
            return a0, a1, b0, b1

        for b in range(NBUF):
            gather(b, b).start()

        @pl.loop(0, nchunk // NBUF)
        def _group(i):
            for b in range(NBUF):
                c = NBUF * i + b
                gather(c, b).wait()
                for e in range(CB):
                    accs = list(halves(b, e * CTX))
                    for r in range(1, CTX):
                        hs = halves(b, e * CTX + r)
                        for k in range(4):
                            accs[k] = accs[k] + hs[k]
                    for k in range(4):
                        pooled_v[c * CB + e, pl.ds(16 * k, 16)] = accs[k]

                @pl.when(c + NBUF < nchunk)
                def _prefetch():
                    gather(c + NBUF, b).start()

        pltpu.sync_copy(pooled_v, out_hbm.at[pl.ds(base, bpw)])

    return pool


def _linear_body(p_ref, wt_ref, b_ref, o_ref):
    acc = jnp.dot(p_ref[...], wt_ref[...], preferred_element_type=jnp.float32)
    o_ref[...] = jax.nn.sigmoid(acc + b_ref[...])


def _linear(pooled, wt8, b8):
    batch = pooled.shape[0]
    blk = 2048
    grid = batch // blk
    return pl.pallas_call(
        _linear_body,
        grid=(grid,),
        in_specs=[
            pl.BlockSpec((blk, EMB), lambda i: (i, 0)),
            pl.BlockSpec((EMB, 8), lambda i: (0, 0)),
            pl.BlockSpec((1, 8), lambda i: (0, 0)),
        ],
        out_specs=pl.BlockSpec((blk, 8), lambda i: (i, 0)),
        out_shape=jax.ShapeDtypeStruct((batch, 8), jnp.float32),
    )(pooled, wt8, b8)


def kernel(inputs, table, W, b):
    ctx, batch = inputs.shape
    vocab = table.shape[0]
    idx_flat = inputs.T.reshape(-1).astype(jnp.int32)
    idx_flat = 4 * (idx_flat % QUARTER) + idx_flat // QUARTER
    table_pk = _repack_table(table.T).reshape(4 * QUARTER, EMB // 2)
    pooled = _make_pool_kernel(batch)(table_pk, idx_flat)
    wt8 = jnp.zeros((EMB, 8), jnp.float32).at[:, :NCLS].set(W.T)
    b8 = jnp.zeros((1, 8), jnp.float32).at[0, :NCLS].set(b)
    out8 = _linear(pooled, wt8, b8)
    return out8[:, :NCLS]


# speedup vs baseline: 1.3638x; 1.1822x over previous
import functools

import jax
import jax.numpy as jnp
from jax import lax
from jax.experimental import pallas as pl
from jax.experimental.pallas import tpu as pltpu
from jax.experimental.pallas import tpu_sc as plsc

EMB = 64
CTX = 50
NCLS = 6
CB = 4
NBUF = 2
TC_VB = 4096


QUARTER = 262144


def _rtne16(t):
    xb = lax.bitcast_convert_type(t, jnp.uint32)
    return xb + jnp.uint32(0x7FFF) + ((xb >> 16) & jnp.uint32(1))


def _transpose_body(r0, r1, r2, r3, out_ref):
    h = EMB // 2
    packs = []
    for ref in (r0, r1, r2, r3):
        r = _rtne16(ref[...])
        word = (r[0:h, :] >> 16) | (r[h:EMB, :] & jnp.uint32(0xFFFF0000))
        packs.append(lax.bitcast_convert_type(word, jnp.int32).T)
    out_ref[...] = jnp.concatenate(packs, axis=1)


def _repack_table(table_t):
    emb, vocab = table_t.shape
    steps = QUARTER // TC_VB
    max_blk = (vocab - 1) // TC_VB

    def make_map(q):
        return lambda j: (0, jnp.minimum(q * steps + j, max_blk))

    return pl.pallas_call(
        _transpose_body,
        grid=(steps,),
        in_specs=[pl.BlockSpec((emb, TC_VB), make_map(q)) for q in range(4)],
        out_specs=pl.BlockSpec((TC_VB, 2 * emb), lambda j: (j, 0)),
        out_shape=jax.ShapeDtypeStruct((QUARTER, 2 * emb), jnp.int32),
    )(table_t, table_t, table_t, table_t)


def _make_pool_kernel(batch):
    info = plsc.get_sparse_core_info()
    nw = info.num_cores * info.num_subcores
    bpw = batch // nw
    rows = CB * CTX
    nchunk = bpw // CB
    mesh = plsc.VectorSubcoreMesh(core_axis_name="c", subcore_axis_name="s")

    @functools.partial(
        pl.kernel,
        out_type=jax.ShapeDtypeStruct((batch, EMB), jnp.float32),
        mesh=mesh,
        scratch_types=[
            pltpu.VMEM((bpw * CTX,), jnp.int32),
            pltpu.VMEM((NBUF, rows, EMB // 2), jnp.int32),
            pltpu.VMEM((bpw, EMB), jnp.float32),
        ] + [pltpu.SemaphoreType.DMA] * NBUF,
        compiler_params=pltpu.CompilerParams(use_tc_tiling_on_sc=False),
    )
    def pool(table_hbm, idx_hbm, out_hbm, idx_v, rows_v, pooled_v, *sems):
        wid = lax.axis_index("s") * info.num_cores + lax.axis_index("c")
        base = wid * bpw
        pltpu.sync_copy(idx_hbm.at[pl.ds(base * CTX, bpw * CTX)], idx_v)

        def gather(c, b):
            return pltpu.make_async_copy(
                table_hbm.at[idx_v.at[pl.ds(c * rows, rows)]],
                rows_v.at[b], sems[b])

        def halves(b, row):
            out = []
            for h in range(2):
                w = rows_v[b, row, pl.ds(16 * h, 16)]
                out.append((lax.bitcast_convert_type(w << 16, jnp.float32),
                            lax.bitcast_convert_type(w & jnp.int32(-65536),
                                                     jnp.float32)))
            (a0, b0), (a1, b1) = out
            return a0, a1, b0, b1

        for b in range(NBUF):
            gather(b, b).start()

        @pl.loop(0, nchunk // NBUF)
        def _group(i):
            for b in range(NBUF):
                c = NBUF * i + b
                gather(c, b).wait()
                for e in range(CB):
                    accs = list(halves(b, e * CTX))
                    for r in range(1, CTX):
                        hs = halves(b, e * CTX + r)
                        for k in range(4):
                            accs[k] = accs[k] + hs[k]
                    for k in range(4):
                        pooled_v[c * CB + e, pl.ds(16 * k, 16)] = accs[k]

                @pl.when(c + NBUF < nchunk)
                def _prefetch():
                    gather(c + NBUF, b).start()

        pltpu.sync_copy(pooled_v, out_hbm.at[pl.ds(base, bpw)])

    return pool


def _linear_body(p_ref, wt_ref, b_ref, o_ref):
    acc = jnp.dot(p_ref[...], wt_ref[...], preferred_element_type=jnp.float32)
    o_ref[...] = jax.nn.sigmoid(acc + b_ref[...])


def _linear(pooled, wt8, b8):
    batch = pooled.shape[0]
    blk = 2048
    grid = batch // blk
    return pl.pallas_call(
        _linear_body,
        grid=(grid,),
        in_specs=[
            pl.BlockSpec((blk, EMB), lambda i: (i, 0)),
            pl.BlockSpec((EMB, 8), lambda i: (0, 0)),
            pl.BlockSpec((1, 8), lambda i: (0, 0)),
        ],
        out_specs=pl.BlockSpec((blk, 8), lambda i: (i, 0)),
        out_shape=jax.ShapeDtypeStruct((batch, 8), jnp.float32),
    )(pooled, wt8, b8)


def kernel(inputs, table, W, b):
    ctx, batch = inputs.shape
    vocab = table.shape[0]
    idx_flat = inputs.T.reshape(-1).astype(jnp.int32)
    idx_flat = 4 * (idx_flat % QUARTER) + idx_flat // QUARTER
    table_pk = _repack_table(table.T).reshape(4 * QUARTER, EMB // 2)
    pooled = _make_pool_kernel(batch)(table_pk, idx_flat)
    wt8 = jnp.zeros((EMB, 8), jnp.float32).at[:, :NCLS].set(W.T)
    b8 = jnp.zeros((1, 8), jnp.float32).at[0, :NCLS].set(b)
    out8 = _linear(pooled, wt8, b8)
    return out8[:, :NCLS]


# speedup vs baseline: 1.3753x; 1.0084x over previous
import functools

import jax
import jax.numpy as jnp
from jax import lax
from jax.experimental import pallas as pl
from jax.experimental.pallas import tpu as pltpu
from jax.experimental.pallas import tpu_sc as plsc

EMB = 64
CTX = 50
NCLS = 6
CB = 4
NBUF = 2
TC_VB = 8192


QUARTER = 262144


def _rtne16(t):
    xb = lax.bitcast_convert_type(t, jnp.uint32)
    return xb + jnp.uint32(0x7FFF) + ((xb >> 16) & jnp.uint32(1))


def _transpose_body(r0, r1, r2, r3, out_ref):
    h = EMB // 2
    packs = []
    for ref in (r0, r1, r2, r3):
        r = _rtne16(ref[...])
        word = (r[0:h, :] >> 16) | (r[h:EMB, :] & jnp.uint32(0xFFFF0000))
        packs.append(lax.bitcast_convert_type(word, jnp.int32).T)
    out_ref[...] = jnp.concatenate(packs, axis=1)


def _repack_table(table_t):
    emb, vocab = table_t.shape
    steps = QUARTER // TC_VB
    max_blk = (vocab - 1) // TC_VB

    def make_map(q):
        return lambda j: (0, jnp.minimum(q * steps + j, max_blk))

    return pl.pallas_call(
        _transpose_body,
        grid=(steps,),
        in_specs=[pl.BlockSpec((emb, TC_VB), make_map(q)) for q in range(4)],
        out_specs=pl.BlockSpec((TC_VB, 2 * emb), lambda j: (j, 0)),
        out_shape=jax.ShapeDtypeStruct((QUARTER, 2 * emb), jnp.int32),
    )(table_t, table_t, table_t, table_t)


def _make_pool_kernel(batch):
    info = plsc.get_sparse_core_info()
    nw = info.num_cores * info.num_subcores
    bpw = batch // nw
    rows = CB * CTX
    nchunk = bpw // CB
    mesh = plsc.VectorSubcoreMesh(core_axis_name="c", subcore_axis_name="s")

    @functools.partial(
        pl.kernel,
        out_type=jax.ShapeDtypeStruct((batch, EMB), jnp.float32),
        mesh=mesh,
        scratch_types=[
            pltpu.VMEM((bpw * CTX,), jnp.int32),
            pltpu.VMEM((NBUF, rows, EMB // 2), jnp.int32),
            pltpu.VMEM((bpw, EMB), jnp.float32),
        ] + [pltpu.SemaphoreType.DMA] * NBUF,
        compiler_params=pltpu.CompilerParams(use_tc_tiling_on_sc=False),
    )
    def pool(table_hbm, idx_hbm, out_hbm, idx_v, rows_v, pooled_v, *sems):
        wid = lax.axis_index("s") * info.num_cores + lax.axis_index("c")
        base = wid * bpw
        pltpu.sync_copy(idx_hbm.at[pl.ds(base * CTX, bpw * CTX)], idx_v)

        def gather(c, b):
            return pltpu.make_async_copy(
                table_hbm.at[idx_v.at[pl.ds(c * rows, rows)]],
                rows_v.at[b], sems[b])

        def halves(b, row):
            out = []
            for h in range(2):
                w = rows_v[b, row, pl.ds(16 * h, 16)]
                out.append((lax.bitcast_convert_type(w << 16, jnp.float32),
                            lax.bitcast_convert_type(w & jnp.int32(-65536),
                                                     jnp.float32)))
            (a0, b0), (a1, b1) = out
            return a0, a1, b0, b1

        for b in range(NBUF):
            gather(b, b).start()

        @pl.loop(0, nchunk // NBUF)
        def _group(i):
            for b in range(NBUF):
                c = NBUF * i + b
                gather(c, b).wait()
                for e in range(CB):
                    accs = list(halves(b, e * CTX))
                    for r in range(1, CTX):
                        hs = halves(b, e * CTX + r)
                        for k in range(4):
                            accs[k] = accs[k] + hs[k]
                    for k in range(4):
                        pooled_v[c * CB + e, pl.ds(16 * k, 16)] = accs[k]

                @pl.when(c + NBUF < nchunk)
                def _prefetch():
                    gather(c + NBUF, b).start()

        pltpu.sync_copy(pooled_v, out_hbm.at[pl.ds(base, bpw)])

    return pool


def _linear_body(p_ref, wt_ref, b_ref, o_ref):
    acc = jnp.dot(p_ref[...], wt_ref[...], preferred_element_type=jnp.float32)
    o_ref[...] = jax.nn.sigmoid(acc + b_ref[...])


def _linear(pooled, wt8, b8):
    batch = pooled.shape[0]
    blk = 2048
    grid = batch // blk
    return pl.pallas_call(
        _linear_body,
        grid=(grid,),
        in_specs=[
            pl.BlockSpec((blk, EMB), lambda i: (i, 0)),
            pl.BlockSpec((EMB, 8), lambda i: (0, 0)),
            pl.BlockSpec((1, 8), lambda i: (0, 0)),
        ],
        out_specs=pl.BlockSpec((blk, 8), lambda i: (i, 0)),
        out_shape=jax.ShapeDtypeStruct((batch, 8), jnp.float32),
    )(pooled, wt8, b8)


def kernel(inputs, table, W, b):
    ctx, batch = inputs.shape
    vocab = table.shape[0]
    idx_flat = inputs.T.reshape(-1).astype(jnp.int32)
    idx_flat = 4 * (idx_flat % QUARTER) + idx_flat // QUARTER
    table_pk = _repack_table(table.T).reshape(4 * QUARTER, EMB // 2)
    pooled = _make_pool_kernel(batch)(table_pk, idx_flat)
    wt8 = jnp.zeros((EMB, 8), jnp.float32).at[:, :NCLS].set(W.T)
    b8 = jnp.zeros((1, 8), jnp.float32).at[0, :NCLS].set(b)
    out8 = _linear(pooled, wt8, b8)
    return out8[:, :NCLS]
